# baseline (speedup 1.0000x reference)
import functools

import jax
import jax.numpy as jnp
from jax import lax
from jax.experimental import pallas as pl
from jax.experimental.pallas import tpu as pltpu
from jax.experimental.pallas import tpu_sc as plsc

NC = 2
NS = 16
NW = NC * NS
D = 32
V = 1000000
RPAD = 1000064
NBLK = RPAD // 128
SLOTS = -(-NBLK // NW)
GRP = 128

_params = pltpu.CompilerParams(needs_layout_passes=False,
                               disable_bounds_checks=True)


@jax.jit
def _relayout(w_t):
    mesh = plsc.VectorSubcoreMesh(core_axis_name="c", subcore_axis_name="s")

    @functools.partial(
        pl.kernel,
        out_type=jax.ShapeDtypeStruct((RPAD // 4, 128), jnp.float32),
        mesh=mesh,
        scratch_types=[
            pltpu.VMEM((D, 128), jnp.float32),
            pltpu.VMEM((D, 128), jnp.float32),
            pltpu.SemaphoreType.DMA,
        ],
        compiler_params=_params,
    )
    def k(wt_hbm, tbl_hbm, in_v, out_v, sem):
        wid = lax.axis_index("s") * NC + lax.axis_index("c")
        lane = lax.iota(jnp.int32, 16)

        @pl.loop(0, SLOTS)
        def slot(s):
            blk = wid + s * NW

            @pl.when(blk < NBLK)
            def _():
                r0 = blk * 128
                for ci in range(4):
                    pltpu.async_copy(
                        wt_hbm.at[pl.ds(ci * 8, 8), pl.ds(r0, 128)],
                        in_v.at[pl.ds(ci * 8, 8)], sem)
                for ci in range(4):
                    pltpu.make_async_copy(
                        wt_hbm.at[pl.ds(0, 8), pl.ds(0, 128)],
                        in_v.at[pl.ds(0, 8)], sem).wait()
                for t in range(32):
                    for m in range(8):
                        rows = lane + 16 * (m % 2)
                        col = jnp.full((16,), 4 * t + m // 2, jnp.int32)
                        out_v[t, pl.ds(16 * m, 16)] = plsc.load_gather(
                            in_v, [rows, col])
                pltpu.sync_copy(out_v, tbl_hbm.at[pl.ds(blk * 32, 32)])

    return k(w_t)


@jax.jit
def _gather(idx_flat, tbl):
    b_total = idx_flat.shape[0]
    b_per_w = b_total // NW
    n_groups = b_per_w // GRP
    mesh = plsc.VectorSubcoreMesh(core_axis_name="c", subcore_axis_name="s")

    @functools.partial(
        pl.kernel,
        out_type=jax.ShapeDtypeStruct((20, D, 16384), jnp.float32),
        mesh=mesh,
        scratch_types=[
            pltpu.VMEM((b_per_w,), jnp.int32),
            pltpu.VMEM((b_per_w,), jnp.int32),
            pltpu.VMEM((2, GRP, 128), jnp.float32),
            pltpu.VMEM((D, GRP), jnp.float32),
            pltpu.SemaphoreType.DMA,
            pltpu.SemaphoreType.DMA,
        ],
        compiler_params=_params,
    )
    def k(idx_hbm, tbl_hbm, out_hbm, idx_v, sup_v, rows_v, out_t,
          gsem0, gsem1):
        wid = lax.axis_index("s") * NC + lax.axis_index("c")
        base = wid * b_per_w
        lane = lax.iota(jnp.int32, 16)
        pltpu.sync_copy(idx_hbm.at[pl.ds(base, b_per_w)], idx_v)

        @pl.loop(0, b_per_w // 16, unroll=8)
        def sup_body(g):
            ids = idx_v[pl.ds(g * 16, 16)]
            sup_v[pl.ds(g * 16, 16)] = lax.shift_right_logical(ids, 2)

        gsems = (gsem0, gsem1)

        def fire(g, buf):
            pltpu.async_copy(
                tbl_hbm.at[sup_v.at[pl.ds(g * GRP, GRP)]],
                rows_v.at[buf], gsems[buf])

        def process(g, buf):
            pltpu.make_async_copy(
                tbl_hbm.at[sup_v.at[pl.ds(0, GRP)]],
                rows_v.at[buf], gsems[buf]).wait()
            rows_g = rows_v.at[buf]
            for m in range(GRP // 16):
                ivec = lane + 16 * m
                qoff = (idx_v[pl.ds(g * GRP + 16 * m, 16)] & 3) * D
                for c in range(D):
                    out_t[c, pl.ds(16 * m, 16)] = plsc.load_gather(
                        rows_g, [ivec, qoff + c])
            k0 = base + g * GRP
            s_idx = k0 // 16384
            b0 = k0 % 16384
            pltpu.sync_copy(out_t, out_hbm.at[s_idx, :, pl.ds(b0, GRP)])

        fire(0, 0)

        @pl.loop(0, n_groups, step=2)
        def pair(g):
            fire(g + 1, 1)
            process(g, 0)

            @pl.when(g + 2 < n_groups)
            def _():
                fire(g + 2, 0)
            process(g + 1, 1)

    return k(idx_flat, tbl)


def kernel(token_ids, weight):
    idx_flat = token_ids.T.reshape(-1)
    w_t = weight.T
    tbl = _relayout(w_t)
    out = _gather(idx_flat, tbl)
    return jnp.transpose(out, (2, 0, 1))


# speedup vs baseline: 1.5248x; 1.5248x over previous
import functools

import jax
import jax.numpy as jnp
from jax import lax
from jax.experimental import pallas as pl
from jax.experimental.pallas import tpu as pltpu
from jax.experimental.pallas import tpu_sc as plsc

NC = 2
NS = 16
NW = NC * NS
D = 32
V = 1000000
RPAD = 1000064
NBLK = RPAD // 128
SLOTS = -(-NBLK // NW)
GRP = 128

_params = pltpu.CompilerParams(needs_layout_passes=False,
                               disable_bounds_checks=True)


@jax.jit
def _relayout(w_t):
    mesh = plsc.VectorSubcoreMesh(core_axis_name="c", subcore_axis_name="s")

    @functools.partial(
        pl.kernel,
        out_type=jax.ShapeDtypeStruct((RPAD // 4, 128), jnp.float32),
        mesh=mesh,
        scratch_types=[
            pltpu.VMEM((D, 128), jnp.float32),
            pltpu.VMEM((D, 128), jnp.float32),
            pltpu.SemaphoreType.DMA,
        ],
        compiler_params=_params,
    )
    def k(wt_hbm, tbl_hbm, in_v, out_v, sem):
        wid = lax.axis_index("s") * NC + lax.axis_index("c")
        lane = lax.iota(jnp.int32, 16)

        @pl.loop(0, SLOTS)
        def slot(s):
            blk = wid + s * NW

            @pl.when(blk < NBLK)
            def _():
                r0 = blk * 128
                for ci in range(4):
                    pltpu.async_copy(
                        wt_hbm.at[pl.ds(ci * 8, 8), pl.ds(r0, 128)],
                        in_v.at[pl.ds(ci * 8, 8)], sem)
                for ci in range(4):
                    pltpu.make_async_copy(
                        wt_hbm.at[pl.ds(0, 8), pl.ds(0, 128)],
                        in_v.at[pl.ds(0, 8)], sem).wait()
                @plsc.parallel_loop(0, 32, unroll=4)
                def tloop(t):
                    for m in range(8):
                        rows = lane + 16 * (m % 2)
                        col = jnp.full((16,), m // 2, jnp.int32) + 4 * t
                        out_v[t, pl.ds(16 * m, 16)] = plsc.load_gather(
                            in_v, [rows, col])
                pltpu.sync_copy(out_v, tbl_hbm.at[pl.ds(blk * 32, 32)])

    return k(w_t)


@jax.jit
def _gather(idx_flat, tbl):
    b_total = idx_flat.shape[0]
    b_per_w = b_total // NW
    n_groups = b_per_w // GRP
    mesh = plsc.VectorSubcoreMesh(core_axis_name="c", subcore_axis_name="s")

    @functools.partial(
        pl.kernel,
        out_type=jax.ShapeDtypeStruct((20, D, 16384), jnp.float32),
        mesh=mesh,
        scratch_types=[
            pltpu.VMEM((b_per_w,), jnp.int32),
            pltpu.VMEM((b_per_w,), jnp.int32),
            pltpu.VMEM((2, GRP, 128), jnp.float32),
            pltpu.VMEM((D, GRP), jnp.float32),
            pltpu.SemaphoreType.DMA,
            pltpu.SemaphoreType.DMA,
        ],
        compiler_params=_params,
    )
    def k(idx_hbm, tbl_hbm, out_hbm, idx_v, sup_v, rows_v, out_t,
          gsem0, gsem1):
        wid = lax.axis_index("s") * NC + lax.axis_index("c")
        base = wid * b_per_w
        lane = lax.iota(jnp.int32, 16)
        pltpu.sync_copy(idx_hbm.at[pl.ds(base, b_per_w)], idx_v)

        @pl.loop(0, b_per_w // 16, unroll=8)
        def sup_body(g):
            ids = idx_v[pl.ds(g * 16, 16)]
            sup_v[pl.ds(g * 16, 16)] = lax.shift_right_logical(ids, 2)

        gsems = (gsem0, gsem1)

        def fire(g, buf):
            pltpu.async_copy(
                tbl_hbm.at[sup_v.at[pl.ds(g * GRP, GRP)]],
                rows_v.at[buf], gsems[buf])

        def process(g, buf):
            pltpu.make_async_copy(
                tbl_hbm.at[sup_v.at[pl.ds(0, GRP)]],
                rows_v.at[buf], gsems[buf]).wait()
            rows_g = rows_v.at[buf]
            @plsc.parallel_loop(0, GRP // 16, unroll=2)
            def mloop(m):
                ivec = lane + 16 * m
                qoff = (idx_v[pl.ds(g * GRP + 16 * m, 16)] & 3) * D
                for c in range(D):
                    out_t[c, pl.ds(16 * m, 16)] = plsc.load_gather(
                        rows_g, [ivec, qoff + c])
            k0 = base + g * GRP
            s_idx = k0 // 16384
            b0 = k0 % 16384
            pltpu.sync_copy(out_t, out_hbm.at[s_idx, :, pl.ds(b0, GRP)])

        fire(0, 0)

        @pl.loop(0, n_groups, step=2)
        def pair(g):
            fire(g + 1, 1)
            process(g, 0)

            @pl.when(g + 2 < n_groups)
            def _():
                fire(g + 2, 0)
            process(g + 1, 1)

    return k(idx_flat, tbl)


def kernel(token_ids, weight):
    idx_flat = token_ids.T.reshape(-1)
    w_t = weight.T
    tbl = _relayout(w_t)
    out = _gather(idx_flat, tbl)
    return jnp.transpose(out, (2, 0, 1))


# speedup vs baseline: 2.1853x; 1.4331x over previous
import functools

import jax
import jax.numpy as jnp
from jax import lax
from jax.experimental import pallas as pl
from jax.experimental.pallas import tpu as pltpu
from jax.experimental.pallas import tpu_sc as plsc

NC = 2
NS = 16
NW = NC * NS
D = 32
V = 1000000
RPAD = 1000064
NBLK = RPAD // 128
SLOTS = -(-NBLK // NW)
GRP = 128

_params = pltpu.CompilerParams(needs_layout_passes=False,
                               disable_bounds_checks=True)


@jax.jit
def _relayout(w_t):
    mesh = plsc.VectorSubcoreMesh(core_axis_name="c", subcore_axis_name="s")

    @functools.partial(
        pl.kernel,
        out_type=jax.ShapeDtypeStruct((RPAD // 4, 128), jnp.float32),
        mesh=mesh,
        scratch_types=[
            pltpu.VMEM((2, D, 128), jnp.float32),
            pltpu.VMEM((2, D, 128), jnp.float32),
            pltpu.SemaphoreType.DMA,
            pltpu.SemaphoreType.DMA,
            pltpu.SemaphoreType.DMA,
            pltpu.SemaphoreType.DMA,
        ],
        compiler_params=_params,
    )
    def k(wt_hbm, tbl_hbm, in_v, out_v, isem0, isem1, osem0, osem1):
        wid = lax.axis_index("s") * NC + lax.axis_index("c")
        lane = lax.iota(jnp.int32, 16)
        isems = (isem0, isem1)
        osems = (osem0, osem1)
        n_uniform = NBLK // NW

        def fire_in(s, buf):
            blk = wid + s * NW
            r0 = blk * 128
            for ci in range(4):
                pltpu.async_copy(
                    wt_hbm.at[pl.ds(ci * 8, 8), pl.ds(r0, 128)],
                    in_v.at[buf, pl.ds(ci * 8, 8)], isems[buf])

        def wait_in(buf):
            for ci in range(4):
                pltpu.make_async_copy(
                    wt_hbm.at[pl.ds(0, 8), pl.ds(0, 128)],
                    in_v.at[buf, pl.ds(0, 8)], isems[buf]).wait()

        def transpose(buf):
            inb = in_v.at[buf]

            @plsc.parallel_loop(0, 32, unroll=4)
            def tloop(t):
                for m in range(8):
                    rows = lane + 16 * (m % 2)
                    col = jnp.full((16,), m // 2, jnp.int32) + 4 * t
                    out_v[buf, t, pl.ds(16 * m, 16)] = plsc.load_gather(
                        inb, [rows, col])

        def fire_out(s, buf):
            blk = wid + s * NW
            pltpu.async_copy(out_v.at[buf], tbl_hbm.at[pl.ds(blk * 32, 32)],
                             osems[buf])

        def wait_out(buf):
            pltpu.make_async_copy(out_v.at[buf],
                                  tbl_hbm.at[pl.ds(0, 32)], osems[buf]).wait()

        fire_in(0, 0)

        @pl.loop(0, n_uniform, step=2)
        def pair(s):
            fire_in(s + 1, 1)
            wait_in(0)

            @pl.when(s >= 2)
            def _():
                wait_out(0)
            transpose(0)
            fire_out(s, 0)

            @pl.when(s + 2 < n_uniform)
            def _():
                fire_in(s + 2, 0)
            wait_in(1)

            @pl.when(s >= 1)
            def _():
                wait_out(1)
            transpose(1)
            fire_out(s + 1, 1)

        wait_out(0)
        wait_out(1)

        @pl.when(wid < NBLK % NW)
        def tail():
            fire_in(n_uniform, 0)
            wait_in(0)
            transpose(0)
            fire_out(n_uniform, 0)
            wait_out(0)

    return k(w_t)


@jax.jit
def _gather(idx_flat, tbl):
    b_total = idx_flat.shape[0]
    b_per_w = b_total // NW
    n_groups = b_per_w // GRP
    mesh = plsc.VectorSubcoreMesh(core_axis_name="c", subcore_axis_name="s")

    @functools.partial(
        pl.kernel,
        out_type=jax.ShapeDtypeStruct((20, D, 16384), jnp.float32),
        mesh=mesh,
        scratch_types=[
            pltpu.VMEM((b_per_w,), jnp.int32),
            pltpu.VMEM((b_per_w,), jnp.int32),
            pltpu.VMEM((2, GRP, 128), jnp.float32),
            pltpu.VMEM((D, GRP), jnp.float32),
            pltpu.SemaphoreType.DMA,
            pltpu.SemaphoreType.DMA,
        ],
        compiler_params=_params,
    )
    def k(idx_hbm, tbl_hbm, out_hbm, idx_v, sup_v, rows_v, out_t,
          gsem0, gsem1):
        wid = lax.axis_index("s") * NC + lax.axis_index("c")
        base = wid * b_per_w
        lane = lax.iota(jnp.int32, 16)
        pltpu.sync_copy(idx_hbm.at[pl.ds(base, b_per_w)], idx_v)

        @pl.loop(0, b_per_w // 16, unroll=8)
        def sup_body(g):
            ids = idx_v[pl.ds(g * 16, 16)]
            sup_v[pl.ds(g * 16, 16)] = lax.shift_right_logical(ids, 2)

        gsems = (gsem0, gsem1)

        def fire(g, buf):
            pltpu.async_copy(
                tbl_hbm.at[sup_v.at[pl.ds(g * GRP, GRP)]],
                rows_v.at[buf], gsems[buf])

        def process(g, buf):
            pltpu.make_async_copy(
                tbl_hbm.at[sup_v.at[pl.ds(0, GRP)]],
                rows_v.at[buf], gsems[buf]).wait()
            rows_g = rows_v.at[buf]
            @plsc.parallel_loop(0, GRP // 16, unroll=2)
            def mloop(m):
                ivec = lane + 16 * m
                qoff = (idx_v[pl.ds(g * GRP + 16 * m, 16)] & 3) * D
                for c in range(D):
                    out_t[c, pl.ds(16 * m, 16)] = plsc.load_gather(
                        rows_g, [ivec, qoff + c])
            k0 = base + g * GRP
            s_idx = k0 // 16384
            b0 = k0 % 16384
            pltpu.sync_copy(out_t, out_hbm.at[s_idx, :, pl.ds(b0, GRP)])

        fire(0, 0)

        @pl.loop(0, n_groups, step=2)
        def pair(g):
            fire(g + 1, 1)
            process(g, 0)

            @pl.when(g + 2 < n_groups)
            def _():
                fire(g + 2, 0)
            process(g + 1, 1)

    return k(idx_flat, tbl)


def kernel(token_ids, weight):
    idx_flat = token_ids.T.reshape(-1)
    w_t = weight.T
    tbl = _relayout(w_t)
    out = _gather(idx_flat, tbl)
    return jnp.transpose(out, (2, 0, 1))
